# Initial kernel scaffold; baseline (speedup 1.0000x reference)
import jax
import jax.numpy as jnp
from jax.experimental import pallas as pl


def kernel(rep, W_v, b_v, W_r, b_r, attn_in_w, attn_in_b, attn_out_w, attn_out_b, W_n, b_n, r_n_idx, pad_v_r):
    raise NotImplementedError("write your pallas kernel here")


# speedup vs baseline: 1.1081x; 1.1081x over previous
import functools
import math

import jax
import jax.numpy as jnp
from jax import lax
from jax.experimental import pallas as pl
from jax.experimental.pallas import tpu as pltpu

_B, _REP = 32, 1024
_NV, _NR, _NN, _MR = 504, 190, 11538, 6
_HID, _NH, _MAXN = 32, 4, 512
_HD = _HID // _NH
_HI = jax.lax.Precision.HIGHEST


def _front_kernel(rep_ref, w_ref, b_ref, o_ref):
    o_ref[...] = (
        jnp.dot(rep_ref[...], w_ref[...],
                preferred_element_type=jnp.float32)
        + b_ref[...]
    )


def _front(rep, w_cat, b_cat):
    n = w_cat.shape[1]
    blk = 1024
    grid = (pl.cdiv(n, blk),)
    return pl.pallas_call(
        _front_kernel,
        grid=grid,
        in_specs=[
            pl.BlockSpec((_B, _REP), lambda j: (0, 0)),
            pl.BlockSpec((_REP, blk), lambda j: (0, j)),
            pl.BlockSpec((1, blk), lambda j: (0, j)),
        ],
        out_specs=pl.BlockSpec((_B, blk), lambda j: (0, j)),
        out_shape=jax.ShapeDtypeStruct((_B, n), jnp.float32),
        compiler_params=pltpu.CompilerParams(
            dimension_semantics=("parallel",)),
        name="front_matmul",
    )(rep, w_cat, b_cat.reshape(1, n))


def _mha_kernel(x0_ref, inw_ref, inb_ref, outw_ref, outb_ref, y_ref,
                x_scr, qkv_scr, a_scr):
    scale = 1.0 / math.sqrt(_HD)
    x_scr[...] = x0_ref[...]
    for li in range(3):
        x2 = x_scr[...].reshape(_B * _NR, _HID)
        qkv = lax.dot_general(
            x2, inw_ref[li], (((1,), (1,)), ((), ())),
            preferred_element_type=jnp.float32) + inb_ref[li].reshape(1, 3 * _HID)
        qkv_scr[...] = qkv.reshape(_B, _NR, 3 * _HID)

        def body(b, c):
            qb = qkv_scr[b]
            outs = []
            for h in range(_NH):
                q = qb[:, h * _HD:(h + 1) * _HD]
                k = qb[:, _HID + h * _HD:_HID + (h + 1) * _HD]
                v = qb[:, 2 * _HID + h * _HD:2 * _HID + (h + 1) * _HD]
                s = lax.dot_general(
                    q, k, (((1,), (1,)), ((), ())),
                    preferred_element_type=jnp.float32) * scale
                m = jnp.max(s, axis=-1, keepdims=True)
                e = jnp.exp(s - m)
                a = e / jnp.sum(e, axis=-1, keepdims=True)
                outs.append(jnp.dot(a, v,
                                    preferred_element_type=jnp.float32))
            a_scr[b] = jnp.concatenate(outs, axis=1)
            return c

        lax.fori_loop(0, _B, body, 0)
        a2 = a_scr[...].reshape(_B * _NR, _HID)
        y2 = lax.dot_general(
            a2, outw_ref[li], (((1,), (1,)), ((), ())),
            preferred_element_type=jnp.float32) + outb_ref[li].reshape(1, _HID)
        x_scr[...] = y2.reshape(_B, _NR, _HID)
    y_ref[...] = x_scr[...]


def _mha(x0, in_w, in_b, out_w, out_b):
    return pl.pallas_call(
        _mha_kernel,
        out_shape=jax.ShapeDtypeStruct((_B, _NR, _HID), jnp.float32),
        scratch_shapes=[
            pltpu.VMEM((_B, _NR, _HID), jnp.float32),
            pltpu.VMEM((_B, _NR, 3 * _HID), jnp.float32),
            pltpu.VMEM((_B, _NR, _HID), jnp.float32),
        ],
        name="mha3",
    )(x0, in_w, in_b, out_w, out_b)


_RBLK = 8
_NRP = 192


def _big_kernel(node_ref, wn_ref, bn_ref, gsrc_ref, idx_ref, idxt_ref,
                rn_ref, marg_ref, rmax_ref, rmaxi_ref,
                gbuf, idx_smem, sem):
    cp = pltpu.make_async_copy(idx_ref, idx_smem, sem)
    cp.start()
    node_blk = node_ref[...]
    rn = jnp.dot(node_blk.reshape(_RBLK * _B, _HID), wn_ref[...],
                 preferred_element_type=jnp.float32)
    rn_ref[...] = (rn + bn_ref[...]).reshape(_RBLK, _B, _NN)
    cp.wait()
    miota = lax.broadcasted_iota(jnp.int32, (_MAXN, _B), 0)
    idxt = idxt_ref[0]
    for rl in range(_RBLK):
        for mi0 in range(0, _MAXN, 64):
            for mi in range(mi0, mi0 + 64):
                n = idx_smem[rl, mi]
                gbuf[mi, :] = gsrc_ref[n, 0]
        node_r = node_blk[rl]
        g = lax.dot_general(
            gbuf[:, :_HID], node_r, (((1,), (1,)), ((), ())),
            preferred_element_type=jnp.float32) + gbuf[:, _HID:_HID + 1]
        mx = jnp.max(g, axis=0, keepdims=True)
        e = jnp.exp(g - mx)
        marg = mx + jnp.log(jnp.sum(e, axis=0, keepdims=True))
        cand = jnp.where(g == mx, miota, _MAXN)
        m_star = jnp.min(cand, axis=0, keepdims=True)
        idxv = jnp.broadcast_to(idxt[:, rl:rl + 1], (_MAXN, _B))
        rmaxi = jnp.sum(jnp.where(miota == m_star, idxv, 0),
                        axis=0, keepdims=True)
        marg_ref[rl] = marg
        rmax_ref[rl] = mx
        rmaxi_ref[rl] = rmaxi


def _big(node_r, w_n, b_n, gsrc, idx_pad, idx_t3):
    grid = (_NRP // _RBLK,)
    out_shapes = [
        jax.ShapeDtypeStruct((_NR, _B, _NN), jnp.float32),
        jax.ShapeDtypeStruct((_NR, 1, _B), jnp.float32),
        jax.ShapeDtypeStruct((_NR, 1, _B), jnp.float32),
        jax.ShapeDtypeStruct((_NR, 1, _B), jnp.int32),
    ]
    small = pl.BlockSpec((_RBLK, 1, _B), lambda j: (j, 0, 0))
    return pl.pallas_call(
        _big_kernel,
        grid=grid,
        in_specs=[
            pl.BlockSpec((_RBLK, _B, _HID), lambda j: (j, 0, 0)),
            pl.BlockSpec((_HID, _NN), lambda j: (0, 0)),
            pl.BlockSpec((1, _NN), lambda j: (0, 0)),
            pl.BlockSpec((_NN, 1, _HID + 1), lambda j: (0, 0, 0)),
            pl.BlockSpec((_RBLK, _MAXN), lambda j: (j, 0)),
            pl.BlockSpec((1, _MAXN, _RBLK), lambda j: (j, 0, 0)),
        ],
        out_specs=[
            pl.BlockSpec((_RBLK, _B, _NN), lambda j: (j, 0, 0)),
            small, small, small,
        ],
        out_shape=out_shapes,
        scratch_shapes=[
            pltpu.VMEM((_MAXN, _HID + 1), jnp.float32),
            pltpu.SMEM((_RBLK, _MAXN), jnp.int32),
            pltpu.SemaphoreType.DMA,
        ],
        compiler_params=pltpu.CompilerParams(
            dimension_semantics=("parallel",),
            vmem_limit_bytes=56 * 1024 * 1024),
        name="rn_potential_gather",
    )(node_r, w_n, b_n.reshape(1, _NN), gsrc, idx_pad, idx_t3)


def _verb_kernel(pad_ref, pad2_ref, rnm_ref, rmx_ref, rmi_ref, vpot_ref,
                 norm_ref, vmax_ref, vmaxi_ref):
    oh = (pad_ref[...] == lax.broadcasted_iota(
        jnp.int32, (_NV * _MR, _NR + 1), 1)).astype(jnp.float32)
    vmif = jnp.dot(oh, rmi_ref[...].astype(jnp.float32), precision=_HI,
                   preferred_element_type=jnp.float32)
    vmaxi_ref[...] = jnp.round(vmif).astype(jnp.int32)

    iota_v = lax.broadcasted_iota(jnp.int32, (_NV, _NR + 1), 1)
    oh6 = jnp.zeros((_NV, _NR + 1), jnp.float32)
    for j in range(_MR):
        oh6 = oh6 + (pad2_ref[:, j:j + 1] == iota_v).astype(jnp.float32)
    vsum = jnp.dot(oh6, rnm_ref[...], precision=_HI,
                   preferred_element_type=jnp.float32)
    vmsum = jnp.dot(oh6, rmx_ref[...], precision=_HI,
                    preferred_element_type=jnp.float32)
    v_marg = vsum + vpot_ref[...]
    vmax_ref[...] = vmsum + vpot_ref[...]
    m = jnp.max(v_marg, axis=0, keepdims=True)
    norm_ref[...] = m + jnp.log(jnp.sum(jnp.exp(v_marg - m), axis=0,
                                        keepdims=True))


def _verb(pad_flat, pad2, rnm_p, rmax_p, rmaxi_p, vpot_t):
    return pl.pallas_call(
        _verb_kernel,
        out_shape=[
            jax.ShapeDtypeStruct((1, _B), jnp.float32),
            jax.ShapeDtypeStruct((_NV, _B), jnp.float32),
            jax.ShapeDtypeStruct((_NV * _MR, _B), jnp.int32),
        ],
        name="verb_stage",
    )(pad_flat, pad2, rnm_p, rmax_p, rmaxi_p, vpot_t)


@jax.jit
def kernel(rep, W_v, b_v, W_r, b_r, attn_in_w, attn_in_b, attn_out_w,
           attn_out_b, W_n, b_n, r_n_idx, pad_v_r):
    w_cat = jnp.concatenate([W_r, W_v], axis=1)
    b_cat = jnp.concatenate([b_r, b_v], axis=0)
    front = _front(rep, w_cat, b_cat)
    v_potential = front[:, _NR * _HID:]
    x0 = front[:, :_NR * _HID].reshape(_B, _NR, _HID)

    node_b = _mha(x0, attn_in_w, attn_in_b, attn_out_w, attn_out_b)
    node_r = jnp.transpose(node_b, (1, 0, 2))

    gsrc = jnp.concatenate([W_n.T, b_n[:, None]], axis=1)
    gsrc = gsrc.reshape(_NN, 1, _HID + 1)
    idx_pad = jnp.concatenate(
        [r_n_idx, jnp.zeros((_NRP - _NR, _MAXN), jnp.int32)], axis=0)
    idx_t3 = idx_pad.T.reshape(_MAXN, _NRP // _RBLK, _RBLK).transpose(1, 0, 2)

    rn_potential, marg3, rmax3, rmaxi3 = _big(node_r, W_n, b_n, gsrc,
                                              idx_pad, idx_t3)

    zf = jnp.zeros((1, _B), jnp.float32)
    zi = jnp.zeros((1, _B), jnp.int32)
    rnm_p = jnp.concatenate([zf, marg3.reshape(_NR, _B)], axis=0)
    rmax_p = jnp.concatenate([zf, rmax3.reshape(_NR, _B)], axis=0)
    rmaxi_p = jnp.concatenate([zi, rmaxi3.reshape(_NR, _B)], axis=0)
    pad_flat = pad_v_r.reshape(_NV * _MR, 1)

    norm2, vmax2, vmaxi2 = _verb(pad_flat, pad_v_r, rnm_p, rmax_p, rmaxi_p,
                                 v_potential.T)
    norm = norm2.reshape(_B)
    v_max = vmax2.T
    vr_maxi_g = vmaxi2.reshape(_NV, _MR, _B).transpose(2, 0, 1)

    return (rep, v_potential, rn_potential, norm, v_max, vr_maxi_g)
